# Initial kernel scaffold; baseline (speedup 1.0000x reference)
import functools

import jax
import jax.numpy as jnp
from jax import lax
from jax.experimental import pallas as pl
from jax.experimental.pallas import tpu as pltpu
from jax.experimental.pallas import tpu_sc as plsc

N = 10000
D = 128
E = 320000
NC, NS = 2, 16
NW = NC * NS
CHUNK = 128
EPT = 10112
E_PAD = EPT * NW
NCHUNK = EPT // CHUNK
N_PAD = 10016
RPT = N_PAD // NS
DEG_W = 16
PAD_DST = N
BN_ROWS = 2000
BN_C = 1.0 / (1.0 + 1e-5) ** 0.5

_MESH = plsc.VectorSubcoreMesh(
    core_axis_name="c", subcore_axis_name="s", num_cores=NC, num_subcores=NS
)


@functools.partial(
    pl.kernel,
    out_type=jax.ShapeDtypeStruct((NC, N_PAD, DEG_W), jnp.float32),
    mesh=_MESH,
    scratch_types=[
        pltpu.VMEM((2, CHUNK), jnp.int32),
        pltpu.VMEM((CHUNK, DEG_W), jnp.float32),
        pltpu.VMEM_SHARED((N_PAD, DEG_W), jnp.float32),
    ],
)
def _deg(edges_hbm, ones_hbm, zeros_hbm, out_hbm, idx_v, ones_v, acc):
    cid = lax.axis_index("c")
    sid = lax.axis_index("s")
    wid = cid * NS + sid
    pltpu.sync_copy(zeros_hbm, acc.at[pl.ds(sid * RPT, RPT)])
    pltpu.sync_copy(ones_hbm, ones_v)
    plsc.subcore_barrier()

    def body(c, carry):
        pltpu.sync_copy(edges_hbm.at[wid * NCHUNK + c], idx_v)
        pltpu.sync_copy(ones_v, acc.at[idx_v.at[1]], add=True)
        return carry

    lax.fori_loop(0, NCHUNK, body, 0)
    plsc.subcore_barrier()
    pltpu.sync_copy(
        acc.at[pl.ds(sid * RPT, RPT)], out_hbm.at[cid, pl.ds(sid * RPT, RPT)]
    )


@functools.partial(
    pl.kernel,
    out_type=jax.ShapeDtypeStruct((NC, N_PAD, D), jnp.float32),
    mesh=_MESH,
    scratch_types=[
        pltpu.VMEM((2, CHUNK), jnp.int32),
        pltpu.VMEM((CHUNK, D), jnp.float32),
        pltpu.SemaphoreType.DMA,
        pltpu.VMEM_SHARED((N_PAD, D), jnp.float32),
    ],
)
def _agg(xs_hbm, edges_hbm, zeros_hbm, out_hbm, idx_v, rows_v, sem, acc):
    cid = lax.axis_index("c")
    sid = lax.axis_index("s")
    wid = cid * NS + sid
    pltpu.sync_copy(zeros_hbm, acc.at[pl.ds(sid * RPT, RPT)])
    plsc.subcore_barrier()

    def body(c, carry):
        pltpu.sync_copy(edges_hbm.at[wid * NCHUNK + c], idx_v)
        pltpu.async_copy(xs_hbm.at[idx_v.at[0]], rows_v, sem).wait()
        pltpu.sync_copy(rows_v, acc.at[idx_v.at[1]], add=True)
        return carry

    lax.fori_loop(0, NCHUNK, body, 0)
    plsc.subcore_barrier()
    pltpu.sync_copy(
        acc.at[pl.ds(sid * RPT, RPT)], out_hbm.at[cid, pl.ds(sid * RPT, RPT)]
    )


def _dinv_of(degp_ref):
    deg = degp_ref[0, :, 0:1] + degp_ref[1, :, 0:1] + 1.0
    return lax.rsqrt(deg)


def _mm1_body(x_ref, w_ref, degp_ref, out_ref):
    dinv = _dinv_of(degp_ref)
    xw = jnp.dot(x_ref[...], w_ref[...], preferred_element_type=jnp.float32)
    out_ref[...] = xw * dinv


def _mm2_body(aggp_ref, xs_ref, degp_ref, b_ref, g_ref, beta_ref, w_ref, out_ref):
    dinv = _dinv_of(degp_ref)
    h = (aggp_ref[0] + aggp_ref[1] + xs_ref[...]) * dinv + b_ref[...]
    h = h * (g_ref[...] * BN_C) + beta_ref[...]
    h = jnp.maximum(h, 0.0)
    out_ref[...] = jnp.dot(h, w_ref[...], preferred_element_type=jnp.float32) * dinv


def _fin_body(aggp_ref, xs_ref, degp_ref, b_ref, out_ref):
    dinv = _dinv_of(degp_ref)
    out_ref[...] = (aggp_ref[0] + aggp_ref[1] + xs_ref[...]) * dinv + b_ref[...]


_GRID = (N // BN_ROWS,)
_ROWS = pl.BlockSpec((BN_ROWS, D), lambda i: (i, 0))
_DEGP = pl.BlockSpec((NC, BN_ROWS, DEG_W), lambda i: (0, i, 0))
_AGGP = pl.BlockSpec((NC, BN_ROWS, D), lambda i: (0, i, 0))
_WMAT = pl.BlockSpec((D, D), lambda i: (0, 0))
_VROW = pl.BlockSpec((1, D), lambda i: (0, 0))
_OUT = jax.ShapeDtypeStruct((N, D), jnp.float32)


def _mm1(x, W, degp):
    return pl.pallas_call(
        _mm1_body, grid=_GRID,
        in_specs=[_ROWS, _WMAT, _DEGP], out_specs=_ROWS, out_shape=_OUT,
    )(x, W, degp)


def _mm2(aggp, xs, degp, b, g, beta, W):
    return pl.pallas_call(
        _mm2_body, grid=_GRID,
        in_specs=[_AGGP, _ROWS, _DEGP, _VROW, _VROW, _VROW, _WMAT],
        out_specs=_ROWS, out_shape=_OUT,
    )(aggp, xs, degp, b, g, beta, W)


def _fin(aggp, xs, degp, b):
    return pl.pallas_call(
        _fin_body, grid=_GRID,
        in_specs=[_AGGP, _ROWS, _DEGP, _VROW], out_specs=_ROWS, out_shape=_OUT,
    )(aggp, xs, degp, b)


def kernel(x, edge_index, W1, b1, W2, b2, gamma, beta):
    src = edge_index[0]
    dst = edge_index[1]
    pad_s = jnp.zeros((E_PAD - E,), jnp.int32)
    pad_d = jnp.full((E_PAD - E,), PAD_DST, jnp.int32)
    edges = jnp.stack([
        jnp.concatenate([src, pad_s]),
        jnp.concatenate([dst, pad_d]),
    ])
    edges = edges.reshape(2, NW * NCHUNK, CHUNK).transpose(1, 0, 2)

    ones_deg = jnp.ones((CHUNK, DEG_W), jnp.float32)
    zeros_deg = jnp.zeros((RPT, DEG_W), jnp.float32)
    zeros_row = jnp.zeros((RPT, D), jnp.float32)
    b1r = b1.reshape(1, D)
    b2r = b2.reshape(1, D)
    gr = gamma.reshape(1, D)
    br = beta.reshape(1, D)

    degp = _deg(edges, ones_deg, zeros_deg)
    xs1 = _mm1(x, W1, degp)
    agg1 = _agg(xs1, edges, zeros_row)
    xs2 = _mm2(agg1, xs1, degp, b1r, gr, br, W2)
    agg2 = _agg(xs2, edges, zeros_row)
    return _fin(agg2, xs2, degp, b2r)


# speedup vs baseline: 10.7754x; 10.7754x over previous
import functools

import jax
import jax.numpy as jnp
from jax import lax
from jax.experimental import pallas as pl
from jax.experimental.pallas import tpu as pltpu
from jax.experimental.pallas import tpu_sc as plsc

N = 10000
D = 128
E = 320000
NC, NS = 2, 16
NW = NC * NS
CHUNK = 128
EPT = 10112
E_PAD = EPT * NW
NCHUNK = EPT // CHUNK
N_PAD = 10240
HROWS = N_PAD // D
RPT = N_PAD // NS
HPT = HROWS // NS
PAD_DST = N
BN_ROWS = 2000
BN_C = 1.0 / (1.0 + 1e-5) ** 0.5


def _sc_mesh():
    return plsc.VectorSubcoreMesh(
        core_axis_name="c", subcore_axis_name="s", num_cores=NC, num_subcores=NS
    )


def _deg_body(edges_hbm, ones_hbm, zeros_hbm, out_hbm, idx_v, ones_v, acc):
    cid = lax.axis_index("c")
    sid = lax.axis_index("s")
    wid = cid * NS + sid
    pltpu.sync_copy(zeros_hbm, acc.at[pl.ds(sid * RPT, RPT)])
    pltpu.sync_copy(ones_hbm, ones_v)
    plsc.subcore_barrier()

    def body(c, carry):
        pltpu.sync_copy(edges_hbm.at[wid * NCHUNK + c], idx_v)
        pltpu.sync_copy(ones_v, acc.at[idx_v.at[1]], add=True)
        return carry

    lax.fori_loop(0, NCHUNK, body, 0)
    plsc.subcore_barrier()
    pltpu.sync_copy(
        acc.at[pl.ds(sid * RPT, RPT)], out_hbm.at[cid, pl.ds(sid * RPT, RPT)]
    )


def _agg_body(xs_hbm, edges_hbm, zeros_hbm, out_hbm, idx_v, rows_v, sem, acc):
    cid = lax.axis_index("c")
    sid = lax.axis_index("s")
    wid = cid * NS + sid
    pltpu.sync_copy(zeros_hbm, acc.at[pl.ds(sid * RPT, RPT)])
    plsc.subcore_barrier()

    def body(c, carry):
        pltpu.sync_copy(edges_hbm.at[wid * NCHUNK + c], idx_v)
        pltpu.async_copy(xs_hbm.at[idx_v.at[0]], rows_v, sem).wait()
        pltpu.sync_copy(rows_v, acc.at[idx_v.at[1]], add=True)
        return carry

    lax.fori_loop(0, NCHUNK, body, 0)
    plsc.subcore_barrier()
    pltpu.sync_copy(
        acc.at[pl.ds(sid * RPT, RPT)], out_hbm.at[cid, pl.ds(sid * RPT, RPT)]
    )


@functools.lru_cache(maxsize=None)
def _sc_kernels():
    deg = pl.kernel(
        _deg_body,
        out_type=jax.ShapeDtypeStruct((NC, N_PAD, D), jnp.float32),
        mesh=_sc_mesh(),
        scratch_types=[
            pltpu.VMEM((2, CHUNK), jnp.int32),
            pltpu.VMEM((CHUNK, D), jnp.float32),
            pltpu.VMEM_SHARED((N_PAD, D), jnp.float32),
        ],
    )
    agg = pl.kernel(
        _agg_body,
        out_type=jax.ShapeDtypeStruct((NC, N_PAD, D), jnp.float32),
        mesh=_sc_mesh(),
        scratch_types=[
            pltpu.VMEM((2, CHUNK), jnp.int32),
            pltpu.VMEM((CHUNK, D), jnp.float32),
            pltpu.SemaphoreType.DMA,
            pltpu.VMEM_SHARED((N_PAD, D), jnp.float32),
        ],
    )
    return deg, agg


def _deg(edges, ones, zeros):
    return _sc_kernels()[0](edges, ones, zeros)


def _agg(xs, edges, zeros):
    return _sc_kernels()[1](xs, edges, zeros)


def _dinv_of(degp_ref):
    deg = degp_ref[0] + degp_ref[1] + 1.0
    return lax.rsqrt(deg)


def _mm1_body(x_ref, w_ref, degp_ref, out_ref):
    dinv = _dinv_of(degp_ref)
    xw = jnp.dot(x_ref[...], w_ref[...], preferred_element_type=jnp.float32)
    out_ref[...] = xw * dinv


def _mm2_body(aggp_ref, xs_ref, degp_ref, b_ref, g_ref, beta_ref, w_ref, out_ref):
    dinv = _dinv_of(degp_ref)
    h = (aggp_ref[0] + aggp_ref[1] + xs_ref[...]) * dinv + b_ref[...]
    h = h * (g_ref[...] * BN_C) + beta_ref[...]
    h = jnp.maximum(h, 0.0)
    out_ref[...] = jnp.dot(h, w_ref[...], preferred_element_type=jnp.float32) * dinv


def _fin_body(aggp_ref, xs_ref, degp_ref, b_ref, out_ref):
    dinv = _dinv_of(degp_ref)
    out_ref[...] = (aggp_ref[0] + aggp_ref[1] + xs_ref[...]) * dinv + b_ref[...]


_GRID = (N // BN_ROWS,)
_ROWS = pl.BlockSpec((BN_ROWS, D), lambda i: (i, 0))
_DEGP = pl.BlockSpec((NC, BN_ROWS, D), lambda i: (0, i, 0))
_AGGP = pl.BlockSpec((NC, BN_ROWS, D), lambda i: (0, i, 0))
_WMAT = pl.BlockSpec((D, D), lambda i: (0, 0))
_VROW = pl.BlockSpec((1, D), lambda i: (0, 0))
_OUT = jax.ShapeDtypeStruct((N, D), jnp.float32)


def _mm1(x, W, degp):
    return pl.pallas_call(
        _mm1_body, grid=_GRID,
        in_specs=[_ROWS, _WMAT, _DEGP], out_specs=_ROWS, out_shape=_OUT,
    )(x, W, degp)


def _mm2(aggp, xs, degp, b, g, beta, W):
    return pl.pallas_call(
        _mm2_body, grid=_GRID,
        in_specs=[_AGGP, _ROWS, _DEGP, _VROW, _VROW, _VROW, _WMAT],
        out_specs=_ROWS, out_shape=_OUT,
    )(aggp, xs, degp, b, g, beta, W)


def _fin(aggp, xs, degp, b):
    return pl.pallas_call(
        _fin_body, grid=_GRID,
        in_specs=[_AGGP, _ROWS, _DEGP, _VROW], out_specs=_ROWS, out_shape=_OUT,
    )(aggp, xs, degp, b)


def kernel(x, edge_index, W1, b1, W2, b2, gamma, beta):
    src = edge_index[0]
    dst = edge_index[1]
    pad_s = jnp.zeros((E_PAD - E,), jnp.int32)
    pad_d = jnp.full((E_PAD - E,), PAD_DST, jnp.int32)
    edges = jnp.stack([
        jnp.concatenate([src, pad_s]),
        jnp.concatenate([dst, pad_d]),
    ])
    edges = edges.reshape(2, NW * NCHUNK, CHUNK).transpose(1, 0, 2)

    ones_row = jnp.ones((CHUNK, D), jnp.float32)
    zeros_row = jnp.zeros((RPT, D), jnp.float32)
    b1r = b1.reshape(1, D)
    b2r = b2.reshape(1, D)
    gr = gamma.reshape(1, D)
    br = beta.reshape(1, D)

    degp = _deg(edges, ones_row, zeros_row)
    xs1 = _mm1(x, W1, degp)
    agg1 = _agg(xs1, edges, zeros_row)
    xs2 = _mm2(agg1, xs1, degp, b1r, gr, br, W2)
    agg2 = _agg(xs2, edges, zeros_row)
    return _fin(agg2, xs2, degp, b2r)
